# Initial kernel scaffold; baseline (speedup 1.0000x reference)
import jax
import jax.numpy as jnp
from jax.experimental import pallas as pl


def kernel(sequence, char_seq, embed_table, elem_table, conv_w):
    raise NotImplementedError("write your pallas kernel here")


# speedup vs baseline: 3.9338x; 3.9338x over previous
import functools

import jax
import jax.numpy as jnp
from jax import lax
from jax.experimental import pallas as pl
from jax.experimental.pallas import tpu as pltpu
from jax.experimental.pallas import tpu_sc as plsc

_CVN = 1000
_VOCAB_PAD = 1024
_MAXSEQ = 20
_EDIM = 32
_FDIM = 64
_NC, _NS = 2, 16
_NW = _NC * _NS
_ITEMS = 51200
_PW = _ITEMS // _NW
_CH = 80
_NSUB = _PW // _CH


def _g_matmul_body(elem_ref, w_ref, out_ref):
    out_ref[0] = jnp.dot(elem_ref[...], w_ref[0], preferred_element_type=jnp.float32)


def _build_g(elem_pad, w_t):
    return pl.pallas_call(
        _g_matmul_body,
        grid=(_MAXSEQ,),
        in_specs=[
            pl.BlockSpec((_VOCAB_PAD, _EDIM), lambda k: (0, 0)),
            pl.BlockSpec((1, _EDIM, _FDIM), lambda k: (k, 0, 0)),
        ],
        out_specs=pl.BlockSpec((1, _VOCAB_PAD, _FDIM), lambda k: (k, 0, 0)),
        out_shape=jax.ShapeDtypeStruct((_MAXSEQ, _VOCAB_PAD, _FDIM), jnp.float32),
    )(elem_pad, w_t)


_SC_MESH = plsc.VectorSubcoreMesh(
    core_axis_name="c", subcore_axis_name="s", num_cores=_NC, num_subcores=_NS
)


@functools.partial(
    pl.kernel,
    out_type=(
        jax.ShapeDtypeStruct((_ITEMS, _EDIM), jnp.float32),
        jax.ShapeDtypeStruct((_ITEMS, _FDIM), jnp.float32),
    ),
    mesh=_SC_MESH,
    scratch_types=[
        pltpu.VMEM((_PW,), jnp.int32),
        pltpu.VMEM((_CH, _FDIM), jnp.float32),
        pltpu.VMEM((_PW, _FDIM), jnp.float32),
        pltpu.VMEM((_CH,), jnp.int32),
        pltpu.VMEM((_CH, _EDIM), jnp.float32),
        pltpu.SemaphoreType.DMA,
        pltpu.SemaphoreType.DMA,
    ],
    compiler_params=pltpu.CompilerParams(use_tc_tiling_on_sc=False),
)
def _sc_embed(seq_idx_hbm, char_idx_hbm, embed_hbm, g_hbm,
              seq_out_hbm, char_out_hbm,
              cidx, rbuf, acc, sidx, srows, gsem, ssem):
    wid = lax.axis_index("s") * _NC + lax.axis_index("c")
    base = pl.multiple_of(wid * _PW, _PW)

    def seq_chunk(c, carry):
        off = pl.multiple_of(base + c * _CH, _CH)
        pltpu.sync_copy(seq_idx_hbm.at[pl.ds(off, _CH)], sidx)
        pltpu.async_copy(embed_hbm.at[sidx], srows, ssem).wait()
        pltpu.sync_copy(srows, seq_out_hbm.at[pl.ds(off, _CH)])
        return carry

    lax.fori_loop(0, _NSUB, seq_chunk, 0)

    def k_pass(k, first):
        koff = pl.multiple_of(k * _ITEMS + base, _CH)
        pltpu.sync_copy(char_idx_hbm.at[pl.ds(koff, _PW)], cidx)

        def sub_step(s, carry):
            soff = pl.multiple_of(s * _CH, _CH)
            pltpu.async_copy(
                g_hbm.at[cidx.at[pl.ds(soff, _CH)]], rbuf, gsem
            ).wait()

            def item_step(i, c2):
                row = soff + i
                for j in range(_FDIM // 16):
                    x = rbuf[i, pl.ds(j * 16, 16)]
                    if first:
                        acc[row, pl.ds(j * 16, 16)] = x
                    else:
                        plsc.addupdate(acc.at[row, pl.ds(j * 16, 16)], x)
                return c2

            lax.fori_loop(0, _CH, item_step, 0)
            return carry

        lax.fori_loop(0, _NSUB, sub_step, 0)
        return 0

    k_pass(0, True)
    lax.fori_loop(1, _MAXSEQ, lambda k, c: k_pass(k, False), 0)
    pltpu.sync_copy(acc, char_out_hbm.at[pl.ds(base, _PW)])


def kernel(sequence, char_seq, embed_table, elem_table, conv_w):
    L, N = sequence.shape
    seq_flat = sequence.reshape(L * N)
    cs = char_seq.reshape(L * N, _MAXSEQ).astype(jnp.int32)
    cs = jnp.where(cs < 0, _CVN, cs)
    cidx = cs.T + (jnp.arange(_MAXSEQ, dtype=jnp.int32) * _VOCAB_PAD)[:, None]
    cidx = cidx.reshape(_MAXSEQ * _ITEMS)
    elem_pad = jnp.pad(elem_table, ((0, _VOCAB_PAD - elem_table.shape[0]), (0, 0)))
    w_t = conv_w.transpose(2, 1, 0)
    g = _build_g(elem_pad, w_t).reshape(_MAXSEQ * _VOCAB_PAD, _FDIM)
    seq_out, char_out = _sc_embed(seq_flat, cidx, embed_table, g)
    return seq_out.reshape(L, N, _EDIM), char_out.reshape(L, N, _FDIM)


# speedup vs baseline: 5.7534x; 1.4626x over previous
import functools

import jax
import jax.numpy as jnp
from jax import lax
from jax.experimental import pallas as pl
from jax.experimental.pallas import tpu as pltpu
from jax.experimental.pallas import tpu_sc as plsc

_CVN = 1000
_VOCAB_PAD = 1024
_MAXSEQ = 20
_EDIM = 32
_FDIM = 64
_NC, _NS = 2, 16
_NW = _NC * _NS
_ITEMS = 51200
_PW = _ITEMS // _NW
_IPB = 80
_NBLK = _PW // _IPB
_RPS = 80
_IPS = _RPS // _MAXSEQ
_NSUB = _IPB // _IPS
_SCH = 80


def _g_matmul_body(elem_ref, w_ref, out_ref):
    out_ref[...] = jnp.dot(elem_ref[...], w_ref[0], preferred_element_type=jnp.float32)


def _build_g(elem_pad, w_t):
    return pl.pallas_call(
        _g_matmul_body,
        grid=(_MAXSEQ,),
        in_specs=[
            pl.BlockSpec((_VOCAB_PAD, _EDIM), lambda k: (0, 0)),
            pl.BlockSpec((1, _EDIM, _FDIM), lambda k: (k, 0, 0)),
        ],
        out_specs=pl.BlockSpec((_VOCAB_PAD, _FDIM), lambda k: (k, 0)),
        out_shape=jax.ShapeDtypeStruct((_MAXSEQ * _VOCAB_PAD, _FDIM), jnp.float32),
    )(elem_pad, w_t)


_SC_MESH = plsc.VectorSubcoreMesh(
    core_axis_name="c", subcore_axis_name="s", num_cores=_NC, num_subcores=_NS
)


@functools.partial(
    pl.kernel,
    out_type=(
        jax.ShapeDtypeStruct((_ITEMS, _EDIM), jnp.float32),
        jax.ShapeDtypeStruct((_ITEMS, _FDIM), jnp.float32),
    ),
    mesh=_SC_MESH,
    scratch_types=[
        pltpu.VMEM((_IPB * _MAXSEQ,), jnp.int32),
        pltpu.VMEM((_IPB * _MAXSEQ,), jnp.int32),
        pltpu.VMEM((2, _RPS, _FDIM), jnp.float32),
        pltpu.VMEM((_IPB, _FDIM), jnp.float32),
        pltpu.VMEM((_SCH,), jnp.int32),
        pltpu.VMEM((_SCH, _EDIM), jnp.float32),
        pltpu.SemaphoreType.DMA,
        pltpu.SemaphoreType.DMA,
        pltpu.SemaphoreType.DMA,
    ],
    compiler_params=pltpu.CompilerParams(use_tc_tiling_on_sc=False),
)
def _sc_embed(seq_idx_hbm, char_hbm, embed_hbm, g_hbm,
              seq_out_hbm, char_out_hbm,
              cidx, offs, rbuf, obuf, sidx, srows, gsem0, gsem1, ssem):
    wid = lax.axis_index("s") * _NC + lax.axis_index("c")
    base = pl.multiple_of(wid * _PW, _PW)

    def seq_chunk(c, carry):
        off = pl.multiple_of(base + c * _SCH, _SCH)
        pltpu.sync_copy(seq_idx_hbm.at[pl.ds(off, _SCH)], sidx)
        pltpu.async_copy(embed_hbm.at[sidx], srows, ssem).wait()
        pltpu.sync_copy(srows, seq_out_hbm.at[pl.ds(off, _SCH)])
        return carry

    lax.fori_loop(0, _PW // _SCH, seq_chunk, 0)

    def offs_step(v, carry):
        j = lax.iota(jnp.int32, 16) + v * 16
        offs[pl.ds(v * 16, 16)] = lax.rem(j, _MAXSEQ) * _VOCAB_PAD
        return carry

    lax.fori_loop(0, _IPB * _MAXSEQ // 16, offs_step, 0)

    def issue(s, buf, sem):
        soff = pl.multiple_of(s * _RPS, _RPS)
        pltpu.async_copy(g_hbm.at[cidx.at[pl.ds(soff, _RPS)]], rbuf.at[buf], sem)

    def drain(buf, sem):
        pltpu.make_async_copy(g_hbm.at[cidx.at[pl.ds(0, _RPS)]], rbuf.at[buf], sem).wait()

    def accum(s, buf):
        for it in range(_IPS):
            accs = [rbuf[buf, it * _MAXSEQ, pl.ds(j * 16, 16)]
                    for j in range(_FDIM // 16)]
            for r in range(1, _MAXSEQ):
                for j in range(_FDIM // 16):
                    accs[j] = accs[j] + rbuf[buf, it * _MAXSEQ + r,
                                             pl.ds(j * 16, 16)]
            row = s * _IPS + it
            for j in range(_FDIM // 16):
                obuf[row, pl.ds(j * 16, 16)] = accs[j]

    def char_block(blk, carry):
        coff = pl.multiple_of((base + blk * _IPB) * _MAXSEQ, _RPS)
        pltpu.sync_copy(char_hbm.at[pl.ds(coff, _IPB * _MAXSEQ)], cidx)

        def idx_step(v, c2):
            cs = cidx[pl.ds(v * 16, 16)]
            cs = jnp.where(cs < 0, _CVN, cs)
            cidx[pl.ds(v * 16, 16)] = cs + offs[pl.ds(v * 16, 16)]
            return c2

        lax.fori_loop(0, _IPB * _MAXSEQ // 16, idx_step, 0)

        issue(0, 0, gsem0)

        def pair_step(pair, c2):
            s0 = pair * 2
            drain(0, gsem0)
            issue(s0 + 1, 1, gsem1)
            accum(s0, 0)
            drain(1, gsem1)

            @pl.when(pair < _NSUB // 2 - 1)
            def _():
                issue(s0 + 2, 0, gsem0)

            accum(s0 + 1, 1)
            return c2

        lax.fori_loop(0, _NSUB // 2, pair_step, 0)
        pltpu.sync_copy(obuf, char_out_hbm.at[pl.ds(base + blk * _IPB, _IPB)])
        return carry

    lax.fori_loop(0, _NBLK, char_block, 0)


def kernel(sequence, char_seq, embed_table, elem_table, conv_w):
    L, N = sequence.shape
    seq_flat = sequence.reshape(L * N)
    char_flat = char_seq.reshape(L * N * _MAXSEQ).astype(jnp.int32)
    elem_pad = jnp.pad(elem_table, ((0, _VOCAB_PAD - elem_table.shape[0]), (0, 0)))
    w_t = conv_w.transpose(2, 1, 0)
    g = _build_g(elem_pad, w_t)
    seq_out, char_out = _sc_embed(seq_flat, char_flat, embed_table, g)
    return seq_out.reshape(L, N, _EDIM), char_out.reshape(L, N, _FDIM)
